# baseline (speedup 1.0000x reference)
import jax
import jax.numpy as jnp
from jax import lax
from jax.experimental import pallas as pl
from jax.experimental.pallas import tpu as pltpu

_CHUNK = 8192
_NBUF = 2
_SUB = 2048


def _router_body(x_hbm, w_ref, probs_hbm, logits_hbm,
                 xbuf, pbuf, lbuf, in_sem, p_sem, l_sem):
    n_tokens = x_hbm.shape[0]
    n_chunks = n_tokens // _CHUNK
    w = w_ref[...]

    def in_copy(chunk, buf):
        return pltpu.make_async_copy(
            x_hbm.at[pl.ds(chunk * _CHUNK, _CHUNK), :],
            xbuf.at[buf],
            in_sem.at[buf],
        )

    for b in range(_NBUF):
        in_copy(b, b).start(priority=b % 2)

    def step(j, carry):
        for k in range(_NBUF):
            i = _NBUF * j + k
            in_copy(i, k).wait()

            for s in range(_CHUNK // _SUB):
                @pl.when(i + s >= 1)
                def _():
                    pltpu.make_async_copy(
                        pbuf, probs_hbm.at[pl.ds(0, _SUB), :], p_sem
                    ).wait()
                    pltpu.make_async_copy(
                        lbuf, logits_hbm.at[pl.ds(0, _SUB), :], l_sem
                    ).wait()

                x = xbuf[k, pl.ds(s * _SUB, _SUB), :]
                logits = jnp.dot(x, w, preferred_element_type=jnp.float32)
                m = jnp.max(logits, axis=-1, keepdims=True)
                e = jnp.exp(logits - m)
                probs = e / jnp.sum(e, axis=-1, keepdims=True)
                pbuf[...] = probs
                lbuf[...] = logits

                off = i * _CHUNK + s * _SUB
                pltpu.make_async_copy(
                    pbuf, probs_hbm.at[pl.ds(off, _SUB), :], p_sem
                ).start()
                pltpu.make_async_copy(
                    lbuf, logits_hbm.at[pl.ds(off, _SUB), :], l_sem
                ).start()

            @pl.when(i + _NBUF < n_chunks)
            def _():
                in_copy(i + _NBUF, k).start(priority=k % 2)

        return carry

    lax.fori_loop(0, n_chunks // _NBUF, step, 0)

    pltpu.make_async_copy(
        pbuf, probs_hbm.at[pl.ds(0, _SUB), :], p_sem
    ).wait()
    pltpu.make_async_copy(
        lbuf, logits_hbm.at[pl.ds(0, _SUB), :], l_sem
    ).wait()


def kernel(inputs, num_experts, w):
    n_tokens, d = inputs.shape
    n_exp = w.shape[1]
    probs, logits = pl.pallas_call(
        _router_body,
        in_specs=[
            pl.BlockSpec(memory_space=pl.ANY),
            pl.BlockSpec(memory_space=pltpu.VMEM),
        ],
        out_specs=[
            pl.BlockSpec(memory_space=pl.ANY),
            pl.BlockSpec(memory_space=pl.ANY),
        ],
        out_shape=[
            jax.ShapeDtypeStruct((n_tokens, n_exp), jnp.float32),
            jax.ShapeDtypeStruct((n_tokens, n_exp), jnp.float32),
        ],
        scratch_shapes=[
            pltpu.VMEM((_NBUF, _CHUNK, d), jnp.float32),
            pltpu.VMEM((_SUB, n_exp), jnp.float32),
            pltpu.VMEM((_SUB, n_exp), jnp.float32),
            pltpu.SemaphoreType.DMA((_NBUF,)),
            pltpu.SemaphoreType.DMA,
            pltpu.SemaphoreType.DMA,
        ],
    )(inputs, w)
    return (probs, logits, 0)


# speedup vs baseline: 1.0491x; 1.0491x over previous
import jax
import jax.numpy as jnp
from jax import lax
from jax.experimental import pallas as pl
from jax.experimental.pallas import tpu as pltpu

_CHUNK = 2048
_NBUF = 4


def _router_body(x_hbm, w_ref, probs_hbm, logits_hbm, xbuf, pbuf, lbuf,
                 in_sem0, in_sem1, in_sem2, in_sem3, p_sem, l_sem):
    n_tokens = x_hbm.shape[0]
    n_chunks = n_tokens // _CHUNK
    w = w_ref[...]
    in_sems = (in_sem0, in_sem1, in_sem2, in_sem3)

    def in_copy(chunk, buf):
        return pltpu.make_async_copy(
            x_hbm.at[pl.ds(chunk * _CHUNK, _CHUNK), :],
            xbuf.at[buf],
            in_sems[buf],
        )

    for b in range(_NBUF):
        in_copy(b, b).start(priority=b % 2)

    def step(j, carry):
        i0 = _NBUF * j
        for k in range(_NBUF):
            i = i0 + k
            in_copy(i, k).wait()

            @pl.when(i >= 1)
            def _():
                pltpu.make_async_copy(
                    pbuf, probs_hbm.at[pl.ds(0, _CHUNK), :], p_sem
                ).wait()
                pltpu.make_async_copy(
                    lbuf, logits_hbm.at[pl.ds(0, _CHUNK), :], l_sem
                ).wait()

            x = xbuf[k]
            logits = jnp.dot(x, w, preferred_element_type=jnp.float32)
            m = jnp.max(logits, axis=-1, keepdims=True)
            e = jnp.exp(logits - m)
            probs = e / jnp.sum(e, axis=-1, keepdims=True)
            pbuf[...] = probs
            lbuf[...] = logits

            pltpu.make_async_copy(
                pbuf, probs_hbm.at[pl.ds(i * _CHUNK, _CHUNK), :], p_sem
            ).start()
            pltpu.make_async_copy(
                lbuf, logits_hbm.at[pl.ds(i * _CHUNK, _CHUNK), :], l_sem
            ).start()

            @pl.when(i + _NBUF < n_chunks)
            def _():
                in_copy(i + _NBUF, k).start(priority=k % 2)

        return carry

    lax.fori_loop(0, n_chunks // _NBUF, step, 0)

    pltpu.make_async_copy(
        pbuf, probs_hbm.at[pl.ds(0, _CHUNK), :], p_sem
    ).wait()
    pltpu.make_async_copy(
        lbuf, logits_hbm.at[pl.ds(0, _CHUNK), :], l_sem
    ).wait()


def kernel(inputs, num_experts, w):
    n_tokens, d = inputs.shape
    n_exp = w.shape[1]
    probs, logits = pl.pallas_call(
        _router_body,
        in_specs=[
            pl.BlockSpec(memory_space=pl.ANY),
            pl.BlockSpec(memory_space=pltpu.VMEM),
        ],
        out_specs=[
            pl.BlockSpec(memory_space=pl.ANY),
            pl.BlockSpec(memory_space=pl.ANY),
        ],
        out_shape=[
            jax.ShapeDtypeStruct((n_tokens, n_exp), jnp.float32),
            jax.ShapeDtypeStruct((n_tokens, n_exp), jnp.float32),
        ],
        scratch_shapes=[
            pltpu.VMEM((_NBUF, _CHUNK, d), jnp.float32),
            pltpu.VMEM((_CHUNK, n_exp), jnp.float32),
            pltpu.VMEM((_CHUNK, n_exp), jnp.float32),
            pltpu.SemaphoreType.DMA,
            pltpu.SemaphoreType.DMA,
            pltpu.SemaphoreType.DMA,
            pltpu.SemaphoreType.DMA,
            pltpu.SemaphoreType.DMA,
            pltpu.SemaphoreType.DMA,
        ],
    )(inputs, w)
    return (probs, logits, 0)


# speedup vs baseline: 1.2309x; 1.1733x over previous
import jax
import jax.numpy as jnp
from jax.experimental import pallas as pl
from jax.experimental.pallas import tpu as pltpu

_BLOCK = 2048


def _router_body(xa_ref, xb_ref, w_ref, pa_ref, la_ref, pb_ref, lb_ref):
    w = w_ref[...]
    for x_ref, p_ref, l_ref in ((xa_ref, pa_ref, la_ref),
                                (xb_ref, pb_ref, lb_ref)):
        x = x_ref[...]
        logits = jnp.dot(x, w, preferred_element_type=jnp.float32)
        m = jnp.max(logits, axis=-1, keepdims=True)
        e = jnp.exp(logits - m)
        probs = e / jnp.sum(e, axis=-1, keepdims=True)
        p_ref[...] = probs
        l_ref[...] = logits


def kernel(inputs, num_experts, w):
    n_tokens, d = inputs.shape
    n_exp = w.shape[1]
    half_blocks = n_tokens // (2 * _BLOCK)

    pa, la, pb, lb = pl.pallas_call(
        _router_body,
        grid=(half_blocks,),
        in_specs=[
            pl.BlockSpec((_BLOCK, d), lambda i: (i, 0)),
            pl.BlockSpec((_BLOCK, d), lambda i: (i + half_blocks, 0)),
            pl.BlockSpec((d, n_exp), lambda i: (0, 0)),
        ],
        out_specs=[
            pl.BlockSpec((_BLOCK, n_exp), lambda i: (i, 0)),
            pl.BlockSpec((_BLOCK, n_exp), lambda i: (i, 0)),
            pl.BlockSpec((_BLOCK, n_exp), lambda i: (i, 0)),
            pl.BlockSpec((_BLOCK, n_exp), lambda i: (i, 0)),
        ],
        out_shape=[
            jax.ShapeDtypeStruct((n_tokens // 2, n_exp), jnp.float32),
            jax.ShapeDtypeStruct((n_tokens // 2, n_exp), jnp.float32),
            jax.ShapeDtypeStruct((n_tokens // 2, n_exp), jnp.float32),
            jax.ShapeDtypeStruct((n_tokens // 2, n_exp), jnp.float32),
        ],
        compiler_params=pltpu.CompilerParams(
            dimension_semantics=("arbitrary",),
        ),
    )(inputs, inputs, w)
    probs = jnp.concatenate([pa, pb], axis=0)
    logits = jnp.concatenate([la, lb], axis=0)
    return (probs, logits, 0)


# speedup vs baseline: 1.3057x; 1.0608x over previous
import jax
import jax.numpy as jnp
from jax import lax
from jax.experimental import pallas as pl
from jax.experimental.pallas import tpu as pltpu

_CHUNK = 1024
_NBUF = 8


def _router_body(x_hbm, w_ref, probs_hbm, logits_hbm,
                 xbuf, pbuf, lbuf, in_sem, p_sem, l_sem):
    n_tokens = x_hbm.shape[0]
    n_chunks = n_tokens // _CHUNK
    w = w_ref[...]

    def in_copy(chunk, buf):
        return pltpu.make_async_copy(
            x_hbm.at[pl.ds(chunk * _CHUNK, _CHUNK), :],
            xbuf.at[buf],
            in_sem.at[buf],
        )

    for b in range(_NBUF):
        in_copy(b, b).start(priority=1)

    def step(j, carry):
        i0 = _NBUF * j
        for k in range(_NBUF):
            i = i0 + k
            in_copy(i, k).wait()

            @pl.when(j >= 1)
            def _():
                pltpu.make_async_copy(
                    pbuf.at[k], probs_hbm.at[pl.ds(0, _CHUNK), :], p_sem.at[k]
                ).wait()
                pltpu.make_async_copy(
                    lbuf.at[k], logits_hbm.at[pl.ds(0, _CHUNK), :], l_sem.at[k]
                ).wait()

            x = xbuf[k]
            logits = jnp.dot(x, w, preferred_element_type=jnp.float32)
            m = jnp.max(logits, axis=-1, keepdims=True)
            e = jnp.exp(logits - m)
            probs = e / jnp.sum(e, axis=-1, keepdims=True)
            pbuf[k] = probs
            lbuf[k] = logits

            pltpu.make_async_copy(
                pbuf.at[k], probs_hbm.at[pl.ds(i * _CHUNK, _CHUNK), :],
                p_sem.at[k]
            ).start()
            pltpu.make_async_copy(
                lbuf.at[k], logits_hbm.at[pl.ds(i * _CHUNK, _CHUNK), :],
                l_sem.at[k]
            ).start()

            @pl.when(i + _NBUF < n_chunks)
            def _():
                in_copy(i + _NBUF, k).start(priority=1)

        return carry

    lax.fori_loop(0, n_chunks // _NBUF, step, 0)

    for k in range(_NBUF):
        pltpu.make_async_copy(
            pbuf.at[k], probs_hbm.at[pl.ds(0, _CHUNK), :], p_sem.at[k]
        ).wait()
        pltpu.make_async_copy(
            lbuf.at[k], logits_hbm.at[pl.ds(0, _CHUNK), :], l_sem.at[k]
        ).wait()


def kernel(inputs, num_experts, w):
    n_tokens, d = inputs.shape
    n_exp = w.shape[1]
    probs, logits = pl.pallas_call(
        _router_body,
        in_specs=[
            pl.BlockSpec(memory_space=pl.ANY),
            pl.BlockSpec(memory_space=pltpu.VMEM),
        ],
        out_specs=[
            pl.BlockSpec(memory_space=pl.ANY),
            pl.BlockSpec(memory_space=pl.ANY),
        ],
        out_shape=[
            jax.ShapeDtypeStruct((n_tokens, n_exp), jnp.float32),
            jax.ShapeDtypeStruct((n_tokens, n_exp), jnp.float32),
        ],
        scratch_shapes=[
            pltpu.VMEM((_NBUF, _CHUNK, d), jnp.float32),
            pltpu.VMEM((_NBUF, _CHUNK, n_exp), jnp.float32),
            pltpu.VMEM((_NBUF, _CHUNK, n_exp), jnp.float32),
            pltpu.SemaphoreType.DMA((_NBUF,)),
            pltpu.SemaphoreType.DMA((_NBUF,)),
            pltpu.SemaphoreType.DMA((_NBUF,)),
        ],
    )(inputs, w)
    return (probs, logits, 0)
